# baseline (speedup 1.0000x reference)
import jax
import jax.numpy as jnp
from jax import lax
from jax.experimental import pallas as pl
from jax.experimental.pallas import tpu as pltpu
from jax.experimental.pallas import tpu_sc as plsc

NUM_PATCHES = 1024
PATCH_DIM = 768
DIM_MODEL = 64
BATCH = 1024

CHUNK = 128
HALF = PATCH_DIM // 2
CPR = HALF // CHUNK
RPW = BATCH // 16
RPB = 8
NBLK = RPW // RPB
CPB = RPB * CPR
NCH = RPW * CPR
LANES = 16
NIV = CHUNK // LANES
NOB = 4


def _body(idx_hbm, tokt_hbm, post_hbm, out_hbm, tok_v,
          ix0, ix1, ob0, ob1, ob2, ob3,
          is0, is1, os0, os1, os2, os3, fs0, fs1, fs2, fs3):
    ixs = (ix0, ix1)
    isems = (is0, is1)
    obufs = (ob0, ob1, ob2, ob3)
    osems = (os0, os1, os2, os3)
    fsems = (fs0, fs1, fs2, fs3)

    h = lax.axis_index("c")
    sub = lax.axis_index("s")
    b0 = sub * RPW
    p0 = h * HALF

    pltpu.sync_copy(tokt_hbm, tok_v)

    def start_idx(slot, blk):
        pltpu.async_copy(
            idx_hbm.at[pl.ds(b0 + blk * RPB, RPB), pl.ds(p0, HALF)],
            ixs[slot], isems[slot])

    def wait_idx(slot):
        pltpu.make_async_copy(idx_hbm.at[pl.ds(0, RPB), pl.ds(0, HALF)],
                              ixs[slot], isems[slot]).wait()

    def start_fill(s, c):
        pltpu.async_copy(
            post_hbm.at[pl.ds(0, DIM_MODEL), pl.ds(p0 + c * CHUNK, CHUNK)],
            obufs[s], fsems[s])

    def wait_fill(s):
        pltpu.make_async_copy(
            post_hbm.at[pl.ds(0, DIM_MODEL), pl.ds(0, CHUNK)],
            obufs[s], fsems[s]).wait()

    def start_out(s, b, c):
        pltpu.async_copy(
            obufs[s],
            out_hbm.at[b, pl.ds(0, DIM_MODEL), pl.ds(p0 + c * CHUNK, CHUNK)],
            osems[s])

    def wait_out(s):
        pltpu.make_async_copy(obufs[s],
                              out_hbm.at[0, pl.ds(0, DIM_MODEL), pl.ds(0, CHUNK)],
                              osems[s]).wait()

    def do_chunk(slot, s, rr, c):
        obuf = obufs[s]
        wait_fill(s)
        ivs = tuple(ixs[slot][rr, pl.ds(c * CHUNK + i * LANES, LANES)]
                    for i in range(NIV))

        @plsc.parallel_loop(0, DIM_MODEL, unroll=4, carry=ivs)
        def _(d, fl):
            for i in range(NIV):
                g = plsc.load_gather(tok_v, [fl[i]])
                plsc.addupdate(obuf.at[d, pl.ds(i * LANES, LANES)], g)
            return tuple(f + NUM_PATCHES for f in fl)

    start_fill(0, 0)
    start_fill(1, 1)
    start_idx(0, 0)
    start_idx(1, 1)

    @pl.loop(0, NBLK // 2)
    def _(g):
        for half_blk in range(2):
            blk = g * 2 + half_blk
            slot = half_blk
            wait_idx(slot)

            @pl.loop(0, CPB // NOB)
            def _(q):
                for k in range(NOB):
                    n = q * NOB + k
                    rr = lax.div(n, CPR)
                    c = lax.rem(n, CPR)
                    do_chunk(slot, k, rr, c)
                    start_out(k, b0 + blk * RPB + rr, c)
                    m = blk * CPB + n + 2
                    s2 = (k + 2) % NOB

                    @pl.when(m >= NOB)
                    def _():
                        wait_out(s2)

                    @pl.when(m < NCH)
                    def _():
                        start_fill(s2, lax.rem(m, CPR))

            @pl.when(g < NBLK // 2 - 1)
            def _():
                start_idx(slot, blk + 2)

    wait_out((NCH - 2) % NOB)
    wait_out((NCH - 1) % NOB)


@jax.jit
def _embed(idx, tokt, post):
    mesh = plsc.VectorSubcoreMesh(core_axis_name="c", subcore_axis_name="s")
    scratch = [
        pltpu.VMEM((DIM_MODEL * NUM_PATCHES,), jnp.float32),
        pltpu.VMEM((RPB, HALF), jnp.int32),
        pltpu.VMEM((RPB, HALF), jnp.int32),
    ] + [pltpu.VMEM((DIM_MODEL, CHUNK), jnp.float32) for _ in range(NOB)] \
      + [pltpu.SemaphoreType.DMA for _ in range(2 + 2 * NOB)]
    return pl.kernel(
        _body,
        out_type=jax.ShapeDtypeStruct((BATCH, DIM_MODEL, PATCH_DIM),
                                      jnp.float32),
        mesh=mesh,
        scratch_types=scratch,
        compiler_params=pltpu.CompilerParams(needs_layout_passes=False,
                                             disable_bounds_checks=True),
    )(idx, tokt, post)


def kernel(inputs, token_table, pos_table):
    idx = inputs.astype(jnp.int32)
    out = _embed(idx, token_table.T.reshape(-1), pos_table.T)
    return out.transpose(0, 2, 1)


# speedup vs baseline: 1.7192x; 1.7192x over previous
import jax
import jax.numpy as jnp
from jax import lax
from jax.experimental import pallas as pl
from jax.experimental.pallas import tpu as pltpu
from jax.experimental.pallas import tpu_sc as plsc

NUM_PATCHES = 1024
PATCH_DIM = 768
DIM_MODEL = 64
BATCH = 1024

CHUNK = 128
HALF = PATCH_DIM // 2
CPR = HALF // CHUNK
RPW = BATCH // 16
RPB = 8
NBLK = RPW // RPB
CPB = RPB * CPR
NCH = RPW * CPR
LANES = 16
NIV = CHUNK // LANES
NOB = 4


def _body(idx_hbm, tokt_hbm, post_hbm, out_hbm, tok_v, pos_v,
          ix0, ix1, ob0, ob1, ob2, ob3,
          is0, is1, os0, os1, os2, os3):
    ixs = (ix0, ix1)
    isems = (is0, is1)
    obufs = (ob0, ob1, ob2, ob3)
    osems = (os0, os1, os2, os3)

    h = lax.axis_index("c")
    sub = lax.axis_index("s")
    b0 = sub * RPW
    p0 = h * HALF

    pltpu.sync_copy(tokt_hbm, tok_v)
    pltpu.sync_copy(post_hbm.at[pl.ds(0, DIM_MODEL), pl.ds(p0, HALF)], pos_v)

    def start_idx(slot, blk):
        pltpu.async_copy(
            idx_hbm.at[pl.ds(b0 + blk * RPB, RPB), pl.ds(p0, HALF)],
            ixs[slot], isems[slot])

    def wait_idx(slot):
        pltpu.make_async_copy(idx_hbm.at[pl.ds(0, RPB), pl.ds(0, HALF)],
                              ixs[slot], isems[slot]).wait()

    def start_out(s, b, c):
        pltpu.async_copy(
            obufs[s],
            out_hbm.at[b, pl.ds(0, DIM_MODEL), pl.ds(p0 + c * CHUNK, CHUNK)],
            osems[s])

    def wait_out(s):
        pltpu.make_async_copy(obufs[s],
                              out_hbm.at[0, pl.ds(0, DIM_MODEL), pl.ds(0, CHUNK)],
                              osems[s]).wait()

    def do_chunk(slot, s, rr, c):
        obuf = obufs[s]
        ivs = tuple(ixs[slot][rr, pl.ds(c * CHUNK + i * LANES, LANES)]
                    for i in range(NIV))

        @plsc.parallel_loop(0, DIM_MODEL, unroll=4, carry=ivs)
        def _(d, fl):
            for i in range(NIV):
                g = plsc.load_gather(tok_v, [fl[i]])
                pv = pos_v[d, pl.ds(c * CHUNK + i * LANES, LANES)]
                obuf[d, pl.ds(i * LANES, LANES)] = g + pv
            return tuple(f + NUM_PATCHES for f in fl)

    start_idx(0, 0)
    start_idx(1, 1)

    @pl.loop(0, NBLK // 2)
    def _(g):
        for half_blk in range(2):
            blk = g * 2 + half_blk
            slot = half_blk
            wait_idx(slot)

            @pl.loop(0, CPB // NOB)
            def _(q):
                for k in range(NOB):
                    n = q * NOB + k
                    rr = lax.div(n, CPR)
                    c = lax.rem(n, CPR)
                    m = blk * CPB + n

                    @pl.when(m >= NOB)
                    def _():
                        wait_out(k)

                    do_chunk(slot, k, rr, c)
                    start_out(k, b0 + blk * RPB + rr, c)

            @pl.when(g < NBLK // 2 - 1)
            def _():
                start_idx(slot, blk + 2)

    for k in range(NOB):
        wait_out(k)


@jax.jit
def _embed(idx, tokt, post):
    mesh = plsc.VectorSubcoreMesh(core_axis_name="c", subcore_axis_name="s")
    scratch = [
        pltpu.VMEM((DIM_MODEL * NUM_PATCHES,), jnp.float32),
        pltpu.VMEM((DIM_MODEL, HALF), jnp.float32),
        pltpu.VMEM((RPB, HALF), jnp.int32),
        pltpu.VMEM((RPB, HALF), jnp.int32),
    ] + [pltpu.VMEM((DIM_MODEL, CHUNK), jnp.float32) for _ in range(NOB)] \
      + [pltpu.SemaphoreType.DMA for _ in range(2 + NOB)]
    return pl.kernel(
        _body,
        out_type=jax.ShapeDtypeStruct((BATCH, DIM_MODEL, PATCH_DIM),
                                      jnp.float32),
        mesh=mesh,
        scratch_types=scratch,
        compiler_params=pltpu.CompilerParams(needs_layout_passes=False,
                                             disable_bounds_checks=True),
    )(idx, tokt, post)


def kernel(inputs, token_table, pos_table):
    idx = inputs.astype(jnp.int32)
    out = _embed(idx, token_table.T.reshape(-1), pos_table.T)
    return out.transpose(0, 2, 1)


# speedup vs baseline: 1.9558x; 1.1377x over previous
import jax
import jax.numpy as jnp
from jax import lax
from jax.experimental import pallas as pl
from jax.experimental.pallas import tpu as pltpu
from jax.experimental.pallas import tpu_sc as plsc

NUM_PATCHES = 1024
PATCH_DIM = 768
DIM_MODEL = 64
BATCH = 1024

LANES = 16
CHUNK = 128
NIV = CHUNK // LANES
ROWS = 2
HALF = PATCH_DIM // 2
CPR = HALF // CHUNK
RPW = BATCH // 16
RPB = 8
NBLK = RPW // RPB
GPB = RPB // ROWS
NCG = (RPW // ROWS) * CPR
NOB = 2


def _body(idx_hbm, tokt_hbm, post_hbm, out_hbm, tok_v, pos_v,
          ix0, ix1, ob0, ob1, is0, is1, os0, os1):
    ixs = (ix0, ix1)
    isems = (is0, is1)
    obufs = (ob0, ob1)
    osems = (os0, os1)

    h = lax.axis_index("c")
    sub = lax.axis_index("s")
    b0 = sub * RPW
    p0 = h * HALF

    pltpu.sync_copy(tokt_hbm, tok_v)
    pltpu.sync_copy(post_hbm.at[pl.ds(0, DIM_MODEL), pl.ds(p0, HALF)], pos_v)

    def start_idx(slot, blk):
        pltpu.async_copy(
            idx_hbm.at[pl.ds(b0 + blk * RPB, RPB), pl.ds(p0, HALF)],
            ixs[slot], isems[slot])

    def wait_idx(slot):
        pltpu.make_async_copy(idx_hbm.at[pl.ds(0, RPB), pl.ds(0, HALF)],
                              ixs[slot], isems[slot]).wait()

    def start_out(s, b, c):
        pltpu.async_copy(
            obufs[s],
            out_hbm.at[pl.ds(b, ROWS), pl.ds(0, DIM_MODEL),
                       pl.ds(p0 + c * CHUNK, CHUNK)],
            osems[s])

    def wait_out(s):
        pltpu.make_async_copy(
            obufs[s],
            out_hbm.at[pl.ds(0, ROWS), pl.ds(0, DIM_MODEL), pl.ds(0, CHUNK)],
            osems[s]).wait()

    def do_group(slot, s, rg, c):
        obuf = obufs[s]
        ivs = tuple(ixs[slot][rg * ROWS + r, pl.ds(c * CHUNK + i * LANES,
                                                   LANES)]
                    for r in range(ROWS) for i in range(NIV))

        @plsc.parallel_loop(0, DIM_MODEL, unroll=2, carry=ivs)
        def _(d, fl):
            for i in range(NIV):
                pv = pos_v[d, pl.ds(c * CHUNK + i * LANES, LANES)]
                for r in range(ROWS):
                    g = plsc.load_gather(tok_v, [fl[r * NIV + i]])
                    obuf[r, d, pl.ds(i * LANES, LANES)] = g + pv
            return tuple(f + NUM_PATCHES for f in fl)

    start_idx(0, 0)
    start_idx(1, 1)

    @pl.loop(0, NBLK // 2)
    def _(g):
        for half_blk in range(2):
            blk = g * 2 + half_blk
            slot = half_blk
            wait_idx(slot)

            for t in range(GPB * CPR):
                rg, c = divmod(t, CPR)
                k = t % NOB
                m = blk * (GPB * CPR) + t

                @pl.when(m >= NOB)
                def _():
                    wait_out(k)

                do_group(slot, k, rg, c)
                start_out(k, b0 + blk * RPB + rg * ROWS, c)

            @pl.when(g < NBLK // 2 - 1)
            def _():
                start_idx(slot, blk + 2)

    for k in range(NOB):
        wait_out(k)


@jax.jit
def _embed(idx, tokt, post):
    mesh = plsc.VectorSubcoreMesh(core_axis_name="c", subcore_axis_name="s")
    scratch = [
        pltpu.VMEM((DIM_MODEL * NUM_PATCHES,), jnp.float32),
        pltpu.VMEM((DIM_MODEL, HALF), jnp.float32),
        pltpu.VMEM((RPB, HALF), jnp.int32),
        pltpu.VMEM((RPB, HALF), jnp.int32),
    ] + [pltpu.VMEM((ROWS, DIM_MODEL, CHUNK), jnp.float32)
         for _ in range(NOB)] \
      + [pltpu.SemaphoreType.DMA for _ in range(2 + NOB)]
    return pl.kernel(
        _body,
        out_type=jax.ShapeDtypeStruct((BATCH, DIM_MODEL, PATCH_DIM),
                                      jnp.float32),
        mesh=mesh,
        scratch_types=scratch,
        compiler_params=pltpu.CompilerParams(needs_layout_passes=False,
                                             disable_bounds_checks=True),
    )(idx, tokt, post)


def kernel(inputs, token_table, pos_table):
    idx = inputs.astype(jnp.int32)
    out = _embed(idx, token_table.T.reshape(-1), pos_table.T)
    return out.transpose(0, 2, 1)
